# baseline (speedup 1.0000x reference)
import math

import jax
import jax.numpy as jnp
from jax import lax
from jax.experimental import pallas as pl
from jax.experimental.pallas import tpu as pltpu
from jax.experimental.pallas import tpu_sc as plsc

VOCAB = 100000
DIM = 128
B = 4
S = 8192
EMBED_SCALE_F = math.sqrt(DIM)

_INFO = plsc.get_sparse_core_info()
NC = _INFO.num_cores
NS = _INFO.num_subcores
NW = NC * NS
P = S // NW
K = 8
CH = P // K
NSLOT = 4
LANES = 16
CVECS = DIM // LANES


def _body(ids_hbm, token_hbm, pos_hbm, out_hbm, idx_v, tok_v, pos_v, gsem, wsem):
    wid = lax.axis_index("s") * NC + lax.axis_index("c")
    pstart = wid * P

    id_cps = [
        pltpu.async_copy(
            ids_hbm.at[b, pl.ds(pstart + k * CH, CH)], idx_v.at[b, k], gsem
        )
        for b in range(B)
        for k in range(K)
    ]
    for cp in id_cps:
        cp.wait()

    def fire_chunk(k):
        s = k % NSLOT
        cps = [
            pltpu.async_copy(token_hbm.at[idx_v.at[b, k]], tok_v.at[s, b], gsem)
            for b in range(B)
        ]
        cps.append(
            pltpu.async_copy(
                pos_hbm.at[pl.ds(pstart + k * CH, CH)], pos_v.at[s], gsem
            )
        )
        return cps

    pending = fire_chunk(0)
    writebacks = []
    for k in range(K):
        s = k % NSLOT
        if k + 1 < K:
            if k >= NSLOT - 1:
                for cp in writebacks[k - NSLOT + 1]:
                    cp.wait()
            nxt = fire_chunk(k + 1)
        for cp in pending:
            cp.wait()

        def compute_row(r, _):
            for c in range(CVECS):
                sl = pl.ds(c * LANES, LANES)
                p = pos_v[s, r, sl]
                for b in range(B):
                    tok_v[s, b, r, sl] = tok_v[s, b, r, sl] * EMBED_SCALE_F + p
            return _

        lax.fori_loop(0, CH, compute_row, None)

        writebacks.append(
            [
                pltpu.async_copy(
                    tok_v.at[s, b], out_hbm.at[b, pl.ds(pstart + k * CH, CH)], wsem
                )
                for b in range(B)
            ]
        )
        if k + 1 < K:
            pending = nxt

    for k in range(max(0, K - NSLOT), K):
        for cp in writebacks[k]:
            cp.wait()


@jax.jit
def kernel(input_ids, token_table, pos_table):
    if input_ids.dtype != jnp.int32:
        input_ids = input_ids.astype(jnp.int32)
    mesh = plsc.VectorSubcoreMesh(core_axis_name="c", subcore_axis_name="s")
    out = pl.kernel(
        _body,
        out_type=jax.ShapeDtypeStruct((B, S, DIM), jnp.float32),
        mesh=mesh,
        scratch_types=[
            pltpu.VMEM((B, K, CH), jnp.int32),
            pltpu.VMEM((NSLOT, B, CH, DIM), jnp.float32),
            pltpu.VMEM((NSLOT, CH, DIM), jnp.float32),
            pltpu.SemaphoreType.DMA,
            pltpu.SemaphoreType.DMA,
        ],
    )(input_ids, token_table, pos_table)
    return out


# speedup vs baseline: 1.0493x; 1.0493x over previous
import math

import jax
import jax.numpy as jnp
from jax import lax
from jax.experimental import pallas as pl
from jax.experimental.pallas import tpu as pltpu
from jax.experimental.pallas import tpu_sc as plsc

VOCAB = 100000
DIM = 128
B = 4
S = 8192
EMBED_SCALE_F = math.sqrt(DIM)

_INFO = plsc.get_sparse_core_info()
NC = _INFO.num_cores
NS = _INFO.num_subcores
NW = NC * NS
P = S // NW
K = 4
CH = P // K
NSLOT = 3
LANES = 16
CVECS = DIM // LANES


def _body(
    ids_hbm, token_hbm, pos_hbm, out_hbm, idx_v, tok_v, pos_v, isem, gsem, wsem
):
    wid = lax.axis_index("s") * NC + lax.axis_index("c")
    pstart = wid * P

    id_cps = [
        pltpu.async_copy(
            ids_hbm.at[b, pl.ds(pstart + k * CH, CH)], idx_v.at[b, k], isem
        )
        for b in range(B)
        for k in range(K)
    ]

    def fire_pos(k):
        s = k % NSLOT
        return pltpu.async_copy(
            pos_hbm.at[pl.ds(pstart + k * CH, CH)], pos_v.at[s], gsem
        )

    def fire_gathers(k):
        s = k % NSLOT
        return [
            pltpu.async_copy(token_hbm.at[idx_v.at[b, k]], tok_v.at[s, b], gsem)
            for b in range(B)
        ]

    pos0 = fire_pos(0)
    for cp in id_cps:
        cp.wait()

    def fire_chunk(k):
        return fire_gathers(k) + [fire_pos(k)]

    pending = fire_gathers(0) + [pos0]
    writebacks = []
    for k in range(K):
        s = k % NSLOT
        if k + 1 < K:
            if k >= NSLOT - 1:
                for cp in writebacks[k - NSLOT + 1]:
                    cp.wait()
            nxt = fire_chunk(k + 1)
        for cp in pending:
            cp.wait()

        def compute_row(r, _):
            for c in range(CVECS):
                sl = pl.ds(c * LANES, LANES)
                p = pos_v[s, r, sl]
                for b in range(B):
                    tok_v[s, b, r, sl] = tok_v[s, b, r, sl] * EMBED_SCALE_F + p
            return _

        lax.fori_loop(0, CH, compute_row, None)

        writebacks.append(
            [
                pltpu.async_copy(
                    tok_v.at[s, b], out_hbm.at[b, pl.ds(pstart + k * CH, CH)], wsem
                )
                for b in range(B)
            ]
        )
        if k + 1 < K:
            pending = nxt

    for k in range(max(0, K - NSLOT), K):
        for cp in writebacks[k]:
            cp.wait()


@jax.jit
def kernel(input_ids, token_table, pos_table):
    if input_ids.dtype != jnp.int32:
        input_ids = input_ids.astype(jnp.int32)
    mesh = plsc.VectorSubcoreMesh(core_axis_name="c", subcore_axis_name="s")
    out = pl.kernel(
        _body,
        out_type=jax.ShapeDtypeStruct((B, S, DIM), jnp.float32),
        mesh=mesh,
        scratch_types=[
            pltpu.VMEM((B, K, CH), jnp.int32),
            pltpu.VMEM((NSLOT, B, CH, DIM), jnp.float32),
            pltpu.VMEM((NSLOT, CH, DIM), jnp.float32),
            pltpu.SemaphoreType.DMA,
            pltpu.SemaphoreType.DMA,
            pltpu.SemaphoreType.DMA,
        ],
    )(input_ids, token_table, pos_table)
    return out
